# baseline (speedup 1.0000x reference)
import functools

import jax
import jax.numpy as jnp
from jax import lax
from jax.experimental import pallas as pl
from jax.experimental.pallas import tpu as pltpu
from jax.experimental.pallas import tpu_sc as plsc

N = 10000
E = 320000
D = 128

NC = 2
NS = 16
NW = NC * NS
EPW = E // NW
CH = 128
NCHUNK = 80
EPW_PAD = NCHUNK * CH
E_PAD = NW * EPW_PAD
N_ACC = N + 8
RPS = 624
RPS_LAST = N - 15 * RPS


def _sc_aggregate(x, src_r, dst_r, zeros_nd):
    @functools.partial(
        pl.kernel,
        out_type=jax.ShapeDtypeStruct((NC, N, D), jnp.float32),
        mesh=plsc.VectorSubcoreMesh(core_axis_name="c", subcore_axis_name="s"),
        scratch_types=[
            pltpu.VMEM((NCHUNK, CH), jnp.int32),
            pltpu.VMEM((1, CH), jnp.int32),
            pltpu.VMEM((1, CH), jnp.int32),
            pltpu.VMEM((CH, D), jnp.float32),
            pltpu.VMEM((CH, D), jnp.float32),
            pltpu.VMEM_SHARED((N_ACC, D), jnp.float32),
            pltpu.SemaphoreType.DMA,
            pltpu.SemaphoreType.DMA,
            pltpu.SemaphoreType.DMA,
            pltpu.SemaphoreType.DMA,
        ],
    )
    def agg(x_hbm, s_hbm, d_hbm, z_hbm, out_hbm, src_v, dst_a, dst_b,
            rows_a, rows_b, acc_sh, sem_g0, sem_g1, sem_i0, sem_i1):
        c = lax.axis_index("c")
        s = lax.axis_index("s")
        wid = s * NC + c
        r0 = s * RPS

        pltpu.sync_copy(s_hbm.at[wid], src_v)
        pltpu.async_copy(d_hbm.at[wid, 0], dst_a, sem_i0)
        pltpu.async_copy(d_hbm.at[wid, 1], dst_b, sem_i1)
        pltpu.async_copy(x_hbm.at[src_v.at[0]], rows_a, sem_g0)
        pltpu.async_copy(x_hbm.at[src_v.at[1]], rows_b, sem_g1)

        @pl.when(s < NS - 1)
        def _():
            pltpu.sync_copy(z_hbm.at[pl.ds(r0, RPS)], acc_sh.at[pl.ds(r0, RPS)])

        @pl.when(s == NS - 1)
        def _():
            pltpu.sync_copy(z_hbm.at[pl.ds(15 * RPS, RPS_LAST)],
                            acc_sh.at[pl.ds(15 * RPS, RPS_LAST)])

        plsc.subcore_barrier()

        def body(k, carry):
            i = 2 * k
            pltpu.make_async_copy(d_hbm.at[wid, i], dst_a, sem_i0).wait()
            pltpu.make_async_copy(x_hbm.at[src_v.at[i]], rows_a, sem_g0).wait()
            pltpu.sync_copy(rows_a, acc_sh.at[dst_a.at[0]], add=True)

            @pl.when(i + 2 < NCHUNK)
            def _():
                pltpu.async_copy(d_hbm.at[wid, i + 2], dst_a, sem_i0)
                pltpu.async_copy(x_hbm.at[src_v.at[i + 2]], rows_a, sem_g0)

            pltpu.make_async_copy(d_hbm.at[wid, i + 1], dst_b, sem_i1).wait()
            pltpu.make_async_copy(x_hbm.at[src_v.at[i + 1]], rows_b,
                                  sem_g1).wait()
            pltpu.sync_copy(rows_b, acc_sh.at[dst_b.at[0]], add=True)

            @pl.when(i + 3 < NCHUNK)
            def _():
                pltpu.async_copy(d_hbm.at[wid, i + 3], dst_b, sem_i1)
                pltpu.async_copy(x_hbm.at[src_v.at[i + 3]], rows_b, sem_g1)

            return carry

        lax.fori_loop(0, NCHUNK // 2, body, 0, unroll=False)
        plsc.subcore_barrier()

        @pl.when(s < NS - 1)
        def _():
            pltpu.sync_copy(acc_sh.at[pl.ds(r0, RPS)],
                            out_hbm.at[c, pl.ds(r0, RPS)])

        @pl.when(s == NS - 1)
        def _():
            pltpu.sync_copy(acc_sh.at[pl.ds(15 * RPS, RPS_LAST)],
                            out_hbm.at[c, pl.ds(15 * RPS, RPS_LAST)])

    return agg(x, src_r, dst_r, zeros_nd)


_BN = 1000


def _tc_layer_body(x_ref, a_ref, w1_ref, b1_ref, w2_ref, b2_ref, o_ref):
    h = x_ref[...] + a_ref[0] + a_ref[1]
    h = jnp.maximum(jnp.dot(h, w1_ref[...], preferred_element_type=jnp.float32)
                    + b1_ref[...], 0.0)
    h = jnp.maximum(jnp.dot(h, w2_ref[...], preferred_element_type=jnp.float32)
                    + b2_ref[...], 0.0)
    o_ref[...] = h


def _tc_final_body(x_ref, a_ref, w1_ref, b1_ref, w2_ref, b2_ref,
                   l1w_ref, l1b_ref, l2w_ref, l2b_ref, o_ref):
    h = x_ref[...] + a_ref[0] + a_ref[1]
    h = jnp.maximum(jnp.dot(h, w1_ref[...], preferred_element_type=jnp.float32)
                    + b1_ref[...], 0.0)
    h = jnp.maximum(jnp.dot(h, w2_ref[...], preferred_element_type=jnp.float32)
                    + b2_ref[...], 0.0)
    h = jnp.maximum(jnp.dot(h, l1w_ref[...], preferred_element_type=jnp.float32)
                    + l1b_ref[...], 0.0)
    o_ref[...] = (jnp.dot(h, l2w_ref[...], preferred_element_type=jnp.float32)
                  + l2b_ref[...])


def _row_spec():
    return pl.BlockSpec((_BN, D), lambda i: (i, 0))


def _agg_spec():
    return pl.BlockSpec((NC, _BN, D), lambda i: (0, i, 0))


def _w_spec():
    return pl.BlockSpec((D, D), lambda i: (0, 0))


def _b_spec():
    return pl.BlockSpec((1, D), lambda i: (0, 0))


def _tc_layer(x, agg, w1, b1, w2, b2):
    return pl.pallas_call(
        _tc_layer_body,
        grid=(N // _BN,),
        in_specs=[_row_spec(), _agg_spec(), _w_spec(), _b_spec(), _w_spec(),
                  _b_spec()],
        out_specs=_row_spec(),
        out_shape=jax.ShapeDtypeStruct((N, D), jnp.float32),
    )(x, agg, w1, b1.reshape(1, D), w2, b2.reshape(1, D))


def _tc_final(x, agg, w1, b1, w2, b2, l1w, l1b, l2w, l2b):
    return pl.pallas_call(
        _tc_final_body,
        grid=(N // _BN,),
        in_specs=[_row_spec(), _agg_spec(), _w_spec(), _b_spec(), _w_spec(),
                  _b_spec(), _w_spec(), _b_spec(), _w_spec(), _b_spec()],
        out_specs=_row_spec(),
        out_shape=jax.ShapeDtypeStruct((N, D), jnp.float32),
    )(x, agg, w1, b1.reshape(1, D), w2, b2.reshape(1, D),
      l1w, l1b.reshape(1, D), l2w, l2b.reshape(1, D))


def kernel(x, edge_index, c1w1, c1b1, c1w2, c1b2, c2w1, c2b1, c2w2, c2b2,
           c3w1, c3b1, c3w2, c3b2, l1w, l1b, l2w, l2b):
    pad = E_PAD - E
    src_r = jnp.concatenate(
        [edge_index[0], jnp.zeros((pad,), jnp.int32)]).reshape(NW, NCHUNK, CH)
    dst_r = jnp.concatenate(
        [edge_index[1], jnp.full((pad,), N, jnp.int32)]).reshape(
            NW, NCHUNK, 1, CH)
    z = jnp.zeros((N, D), jnp.float32)
    a = _sc_aggregate(x, src_r, dst_r, z)
    h = _tc_layer(x, a, c1w1, c1b1, c1w2, c1b2)
    a = _sc_aggregate(h, src_r, dst_r, z)
    h = _tc_layer(h, a, c2w1, c2b1, c2w2, c2b2)
    a = _sc_aggregate(h, src_r, dst_r, z)
    return _tc_final(h, a, c3w1, c3b1, c3w2, c3b2, l1w, l1b, l2w, l2b)


# speedup vs baseline: 1.2104x; 1.2104x over previous
import functools

import jax
import jax.numpy as jnp
from jax import lax
from jax.experimental import pallas as pl
from jax.experimental.pallas import tpu as pltpu
from jax.experimental.pallas import tpu_sc as plsc

N = 10000
E = 320000
D = 128

NC = 2
NS = 16
NW = NC * NS
EPW = E // NW
CH = 128
NCHUNK = 80
EPW_PAD = NCHUNK * CH
E_PAD = NW * EPW_PAD
N_SINK = 256
N_ACC = N + N_SINK
RPS = 624
RPS_LAST = N - 15 * RPS


def _sc_aggregate(x, src_r, dst_r, zeros_nd):
    @functools.partial(
        pl.kernel,
        out_type=jax.ShapeDtypeStruct((NC, N, D), jnp.float32),
        mesh=plsc.VectorSubcoreMesh(core_axis_name="c", subcore_axis_name="s"),
        scratch_types=[
            pltpu.VMEM((NCHUNK, CH), jnp.int32),
            pltpu.VMEM((1, CH), jnp.int32),
            pltpu.VMEM((1, CH), jnp.int32),
            pltpu.VMEM((CH, D), jnp.float32),
            pltpu.VMEM((CH, D), jnp.float32),
            pltpu.VMEM_SHARED((N_ACC, D), jnp.float32),
            pltpu.SemaphoreType.DMA,
            pltpu.SemaphoreType.DMA,
            pltpu.SemaphoreType.DMA,
            pltpu.SemaphoreType.DMA,
        ],
    )
    def agg(x_hbm, s_hbm, d_hbm, z_hbm, out_hbm, src_v, dst_a, dst_b,
            rows_a, rows_b, acc_sh, sem_g0, sem_g1, sem_i0, sem_i1):
        c = lax.axis_index("c")
        s = lax.axis_index("s")
        wid = s * NC + c
        r0 = s * RPS

        pltpu.sync_copy(s_hbm.at[wid], src_v)
        pltpu.async_copy(d_hbm.at[wid, 0], dst_a, sem_i0)
        pltpu.async_copy(d_hbm.at[wid, 1], dst_b, sem_i1)
        pltpu.async_copy(x_hbm.at[src_v.at[0]], rows_a, sem_g0)
        pltpu.async_copy(x_hbm.at[src_v.at[1]], rows_b, sem_g1)

        @pl.when(s < NS - 1)
        def _():
            pltpu.sync_copy(z_hbm.at[pl.ds(r0, RPS)], acc_sh.at[pl.ds(r0, RPS)])

        @pl.when(s == NS - 1)
        def _():
            pltpu.sync_copy(z_hbm.at[pl.ds(15 * RPS, RPS_LAST)],
                            acc_sh.at[pl.ds(15 * RPS, RPS_LAST)])

        plsc.subcore_barrier()

        def body(k, carry):
            i = 2 * k
            pltpu.make_async_copy(d_hbm.at[wid, i], dst_a, sem_i0).wait()
            pltpu.make_async_copy(x_hbm.at[src_v.at[i]], rows_a, sem_g0).wait()
            pltpu.sync_copy(rows_a, acc_sh.at[dst_a.at[0]], add=True)

            @pl.when(i + 2 < NCHUNK)
            def _():
                pltpu.async_copy(d_hbm.at[wid, i + 2], dst_a, sem_i0)
                pltpu.async_copy(x_hbm.at[src_v.at[i + 2]], rows_a, sem_g0)

            pltpu.make_async_copy(d_hbm.at[wid, i + 1], dst_b, sem_i1).wait()
            pltpu.make_async_copy(x_hbm.at[src_v.at[i + 1]], rows_b,
                                  sem_g1).wait()
            pltpu.sync_copy(rows_b, acc_sh.at[dst_b.at[0]], add=True)

            @pl.when(i + 3 < NCHUNK)
            def _():
                pltpu.async_copy(d_hbm.at[wid, i + 3], dst_b, sem_i1)
                pltpu.async_copy(x_hbm.at[src_v.at[i + 3]], rows_b, sem_g1)

            return carry

        lax.fori_loop(0, NCHUNK // 2, body, 0, unroll=False)
        plsc.subcore_barrier()

        @pl.when(s < NS - 1)
        def _():
            pltpu.sync_copy(acc_sh.at[pl.ds(r0, RPS)],
                            out_hbm.at[c, pl.ds(r0, RPS)])

        @pl.when(s == NS - 1)
        def _():
            pltpu.sync_copy(acc_sh.at[pl.ds(15 * RPS, RPS_LAST)],
                            out_hbm.at[c, pl.ds(15 * RPS, RPS_LAST)])

    return agg(x, src_r, dst_r, zeros_nd)


_BN = 1000


def _tc_layer_body(x_ref, a_ref, w1_ref, b1_ref, w2_ref, b2_ref, o_ref):
    h = x_ref[...] + a_ref[0] + a_ref[1]
    h = jnp.maximum(jnp.dot(h, w1_ref[...], preferred_element_type=jnp.float32)
                    + b1_ref[...], 0.0)
    h = jnp.maximum(jnp.dot(h, w2_ref[...], preferred_element_type=jnp.float32)
                    + b2_ref[...], 0.0)
    o_ref[...] = h


def _tc_final_body(x_ref, a_ref, w1_ref, b1_ref, w2_ref, b2_ref,
                   l1w_ref, l1b_ref, l2w_ref, l2b_ref, o_ref):
    h = x_ref[...] + a_ref[0] + a_ref[1]
    h = jnp.maximum(jnp.dot(h, w1_ref[...], preferred_element_type=jnp.float32)
                    + b1_ref[...], 0.0)
    h = jnp.maximum(jnp.dot(h, w2_ref[...], preferred_element_type=jnp.float32)
                    + b2_ref[...], 0.0)
    h = jnp.maximum(jnp.dot(h, l1w_ref[...], preferred_element_type=jnp.float32)
                    + l1b_ref[...], 0.0)
    o_ref[...] = (jnp.dot(h, l2w_ref[...], preferred_element_type=jnp.float32)
                  + l2b_ref[...])


def _row_spec():
    return pl.BlockSpec((_BN, D), lambda i: (i, 0))


def _agg_spec():
    return pl.BlockSpec((NC, _BN, D), lambda i: (0, i, 0))


def _w_spec():
    return pl.BlockSpec((D, D), lambda i: (0, 0))


def _b_spec():
    return pl.BlockSpec((1, D), lambda i: (0, 0))


def _tc_layer(x, agg, w1, b1, w2, b2):
    return pl.pallas_call(
        _tc_layer_body,
        grid=(N // _BN,),
        in_specs=[_row_spec(), _agg_spec(), _w_spec(), _b_spec(), _w_spec(),
                  _b_spec()],
        out_specs=_row_spec(),
        out_shape=jax.ShapeDtypeStruct((N, D), jnp.float32),
    )(x, agg, w1, b1.reshape(1, D), w2, b2.reshape(1, D))


def _tc_final(x, agg, w1, b1, w2, b2, l1w, l1b, l2w, l2b):
    return pl.pallas_call(
        _tc_final_body,
        grid=(N // _BN,),
        in_specs=[_row_spec(), _agg_spec(), _w_spec(), _b_spec(), _w_spec(),
                  _b_spec(), _w_spec(), _b_spec(), _w_spec(), _b_spec()],
        out_specs=_row_spec(),
        out_shape=jax.ShapeDtypeStruct((N, D), jnp.float32),
    )(x, agg, w1, b1.reshape(1, D), w2, b2.reshape(1, D),
      l1w, l1b.reshape(1, D), l2w, l2b.reshape(1, D))


def kernel(x, edge_index, c1w1, c1b1, c1w2, c1b2, c2w1, c2b1, c2w2, c2b2,
           c3w1, c3b1, c3w2, c3b2, l1w, l1b, l2w, l2b):
    ppw = EPW_PAD - EPW
    src_r = jnp.concatenate(
        [edge_index[0].reshape(NW, EPW),
         jnp.zeros((NW, ppw), jnp.int32)], axis=1).reshape(NW, NCHUNK, CH)
    sink = (N + jnp.arange(ppw, dtype=jnp.int32) % N_SINK)[None, :]
    dst_r = jnp.concatenate(
        [edge_index[1].reshape(NW, EPW),
         jnp.broadcast_to(sink, (NW, ppw))], axis=1).reshape(NW, NCHUNK, 1, CH)
    z = jnp.zeros((N, D), jnp.float32)
    a = _sc_aggregate(x, src_r, dst_r, z)
    h = _tc_layer(x, a, c1w1, c1b1, c1w2, c1b2)
    a = _sc_aggregate(h, src_r, dst_r, z)
    h = _tc_layer(h, a, c2w1, c2b1, c2w2, c2b2)
    a = _sc_aggregate(h, src_r, dst_r, z)
    return _tc_final(h, a, c3w1, c3b1, c3w2, c3b2, l1w, l1b, l2w, l2b)


# speedup vs baseline: 3.8245x; 3.1595x over previous
import functools

import jax
import jax.numpy as jnp
from jax import lax
from jax.experimental import pallas as pl
from jax.experimental.pallas import tpu as pltpu
from jax.experimental.pallas import tpu_sc as plsc

N = 10000
E = 320000
D = 128

NC = 2
NS = 16
NW = NC * NS
EPW = E // NW
CH = 128
NCHUNK = 80
EPW_PAD = NCHUNK * CH
E_PAD = NW * EPW_PAD
N_SINK = 256
N_ACC = N + N_SINK
RPS = 624
RPS_LAST = N - 15 * RPS


def _sc_aggregate(x, src_r, dst_r, zeros_nd):
    @functools.partial(
        pl.kernel,
        out_type=jax.ShapeDtypeStruct((NC, N, D), jnp.float32),
        mesh=plsc.VectorSubcoreMesh(core_axis_name="c", subcore_axis_name="s"),
        scratch_types=[
            pltpu.VMEM((NCHUNK, CH), jnp.int32),
            pltpu.VMEM((1, CH), jnp.int32),
            pltpu.VMEM((1, CH), jnp.int32),
            pltpu.VMEM((CH, D), jnp.float32),
            pltpu.VMEM((CH, D), jnp.float32),
            pltpu.VMEM_SHARED((N_ACC, D), jnp.float32),
            pltpu.SemaphoreType.DMA,
            pltpu.SemaphoreType.DMA,
            pltpu.SemaphoreType.DMA,
            pltpu.SemaphoreType.DMA,
        ],
    )
    def agg(x_hbm, s_hbm, d_hbm, z_hbm, out_hbm, src_v, dst_a, dst_b,
            rows_a, rows_b, acc_sh, sem_g0, sem_g1, sem_i0, sem_i1):
        c = lax.axis_index("c")
        s = lax.axis_index("s")
        wid = s * NC + c
        r0 = s * RPS

        pltpu.sync_copy(s_hbm.at[wid], src_v)
        pltpu.async_copy(d_hbm.at[wid, 0], dst_a, sem_i0)
        pltpu.async_copy(d_hbm.at[wid, 1], dst_b, sem_i1)
        pltpu.async_copy(x_hbm.at[src_v.at[0]], rows_a, sem_g0)
        pltpu.async_copy(x_hbm.at[src_v.at[1]], rows_b, sem_g1)

        @pl.when(s < NS - 1)
        def _():
            pltpu.sync_copy(z_hbm.at[pl.ds(r0, RPS)], acc_sh.at[pl.ds(r0, RPS)])

        @pl.when(s == NS - 1)
        def _():
            pltpu.sync_copy(z_hbm.at[pl.ds(15 * RPS, RPS_LAST)],
                            acc_sh.at[pl.ds(15 * RPS, RPS_LAST)])

        plsc.subcore_barrier()

        def body(k, carry):
            i = 2 * k
            pltpu.make_async_copy(d_hbm.at[wid, i], dst_a, sem_i0).wait()
            pltpu.make_async_copy(x_hbm.at[src_v.at[i]], rows_a, sem_g0).wait()
            pltpu.sync_copy(rows_a, acc_sh.at[dst_a.at[0]], add=True)

            @pl.when(i + 2 < NCHUNK)
            def _():
                pltpu.async_copy(d_hbm.at[wid, i + 2], dst_a, sem_i0)
                pltpu.async_copy(x_hbm.at[src_v.at[i + 2]], rows_a, sem_g0)

            pltpu.make_async_copy(d_hbm.at[wid, i + 1], dst_b, sem_i1).wait()
            pltpu.make_async_copy(x_hbm.at[src_v.at[i + 1]], rows_b,
                                  sem_g1).wait()
            pltpu.sync_copy(rows_b, acc_sh.at[dst_b.at[0]], add=True)

            @pl.when(i + 3 < NCHUNK)
            def _():
                pltpu.async_copy(d_hbm.at[wid, i + 3], dst_b, sem_i1)
                pltpu.async_copy(x_hbm.at[src_v.at[i + 3]], rows_b, sem_g1)

            return carry

        lax.fori_loop(0, NCHUNK // 2, body, 0, unroll=False)
        plsc.subcore_barrier()

        @pl.when(s < NS - 1)
        def _():
            pltpu.sync_copy(acc_sh.at[pl.ds(r0, RPS)],
                            out_hbm.at[c, pl.ds(r0, RPS)])

        @pl.when(s == NS - 1)
        def _():
            pltpu.sync_copy(acc_sh.at[pl.ds(15 * RPS, RPS_LAST)],
                            out_hbm.at[c, pl.ds(15 * RPS, RPS_LAST)])

    return agg(x, src_r, dst_r, zeros_nd)


_BN = 1000


def _tc_layer_body(x_ref, a_ref, w1_ref, b1_ref, w2_ref, b2_ref, o_ref):
    h = x_ref[...] + a_ref[0] + a_ref[1]
    h = jnp.maximum(jnp.dot(h, w1_ref[...], preferred_element_type=jnp.float32)
                    + b1_ref[...], 0.0)
    h = jnp.maximum(jnp.dot(h, w2_ref[...], preferred_element_type=jnp.float32)
                    + b2_ref[...], 0.0)
    o_ref[...] = h


def _tc_final_body(x_ref, a_ref, w1_ref, b1_ref, w2_ref, b2_ref,
                   l1w_ref, l1b_ref, l2w_ref, l2b_ref, o_ref):
    h = x_ref[...] + a_ref[0] + a_ref[1]
    h = jnp.maximum(jnp.dot(h, w1_ref[...], preferred_element_type=jnp.float32)
                    + b1_ref[...], 0.0)
    h = jnp.maximum(jnp.dot(h, w2_ref[...], preferred_element_type=jnp.float32)
                    + b2_ref[...], 0.0)
    h = jnp.maximum(jnp.dot(h, l1w_ref[...], preferred_element_type=jnp.float32)
                    + l1b_ref[...], 0.0)
    o_ref[...] = (jnp.dot(h, l2w_ref[...], preferred_element_type=jnp.float32)
                  + l2b_ref[...])


def _row_spec():
    return pl.BlockSpec((_BN, D), lambda i: (i, 0))


def _agg_spec():
    return pl.BlockSpec((NC, _BN, D), lambda i: (0, i, 0))


def _w_spec():
    return pl.BlockSpec((D, D), lambda i: (0, 0))


def _b_spec():
    return pl.BlockSpec((1, D), lambda i: (0, 0))


def _tc_layer(x, agg, w1, b1, w2, b2):
    return pl.pallas_call(
        _tc_layer_body,
        grid=(N // _BN,),
        in_specs=[_row_spec(), _agg_spec(), _w_spec(), _b_spec(), _w_spec(),
                  _b_spec()],
        out_specs=_row_spec(),
        out_shape=jax.ShapeDtypeStruct((N, D), jnp.float32),
    )(x, agg, w1, b1.reshape(1, D), w2, b2.reshape(1, D))


def _tc_final(x, agg, w1, b1, w2, b2, l1w, l1b, l2w, l2b):
    return pl.pallas_call(
        _tc_final_body,
        grid=(N // _BN,),
        in_specs=[_row_spec(), _agg_spec(), _w_spec(), _b_spec(), _w_spec(),
                  _b_spec(), _w_spec(), _b_spec(), _w_spec(), _b_spec()],
        out_specs=_row_spec(),
        out_shape=jax.ShapeDtypeStruct((N, D), jnp.float32),
    )(x, agg, w1, b1.reshape(1, D), w2, b2.reshape(1, D),
      l1w, l1b.reshape(1, D), l2w, l2b.reshape(1, D))


def kernel(x, edge_index, c1w1, c1b1, c1w2, c1b2, c2w1, c2b1, c2w2, c2b2,
           c3w1, c3b1, c3w2, c3b2, l1w, l1b, l2w, l2b):
    ppw = EPW_PAD - EPW
    dummy_src = (jnp.arange(ppw, dtype=jnp.int32) * 41) % N
    src_r = jnp.concatenate(
        [edge_index[0].reshape(NW, EPW),
         jnp.broadcast_to(dummy_src[None, :], (NW, ppw))],
        axis=1).reshape(NW, NCHUNK, CH)
    sink = (N + jnp.arange(ppw, dtype=jnp.int32) % N_SINK)[None, :]
    dst_r = jnp.concatenate(
        [edge_index[1].reshape(NW, EPW),
         jnp.broadcast_to(sink, (NW, ppw))], axis=1).reshape(NW, NCHUNK, 1, CH)
    z = jnp.zeros((N, D), jnp.float32)
    a = _sc_aggregate(x, src_r, dst_r, z)
    h = _tc_layer(x, a, c1w1, c1b1, c1w2, c1b2)
    a = _sc_aggregate(h, src_r, dst_r, z)
    h = _tc_layer(h, a, c2w1, c2b1, c2w2, c2b2)
    a = _sc_aggregate(h, src_r, dst_r, z)
    return _tc_final(h, a, c3w1, c3b1, c3w2, c3b2, l1w, l1b, l2w, l2b)


# speedup vs baseline: 3.8382x; 1.0036x over previous
import functools

import jax
import jax.numpy as jnp
from jax import lax
from jax.experimental import pallas as pl
from jax.experimental.pallas import tpu as pltpu
from jax.experimental.pallas import tpu_sc as plsc

N = 10000
E = 320000
D = 128

NC = 2
NS = 16
NW = NC * NS
EPW = E // NW
CH = 128
NCHUNK = 80
EPW_PAD = NCHUNK * CH
E_PAD = NW * EPW_PAD
N_SINK = 256
N_ACC = N + N_SINK
RPS = 624
RPS_LAST = N - 15 * RPS


def _sc_aggregate(x, src_r, dst_r, zeros_nd):
    @functools.partial(
        pl.kernel,
        out_type=jax.ShapeDtypeStruct((NC, N, D), jnp.float32),
        mesh=plsc.VectorSubcoreMesh(core_axis_name="c", subcore_axis_name="s"),
        scratch_types=[
            pltpu.VMEM((NCHUNK, CH), jnp.int32),
            pltpu.VMEM((1, CH), jnp.int32),
            pltpu.VMEM((1, CH), jnp.int32),
            pltpu.VMEM((CH, D), jnp.float32),
            pltpu.VMEM((CH, D), jnp.float32),
            pltpu.VMEM_SHARED((N_ACC, D), jnp.float32),
            pltpu.SemaphoreType.DMA,
            pltpu.SemaphoreType.DMA,
            pltpu.SemaphoreType.DMA,
            pltpu.SemaphoreType.DMA,
        ],
    )
    def agg(x_hbm, s_hbm, d_hbm, z_hbm, out_hbm, src_v, dst_a, dst_b,
            rows_a, rows_b, acc_sh, sem_g0, sem_g1, sem_i0, sem_i1):
        c = lax.axis_index("c")
        s = lax.axis_index("s")
        wid = s * NC + c
        r0 = s * RPS

        pltpu.sync_copy(s_hbm.at[wid], src_v)
        pltpu.async_copy(d_hbm.at[wid, 0], dst_a, sem_i0)
        pltpu.async_copy(d_hbm.at[wid, 1], dst_b, sem_i1)
        pltpu.async_copy(x_hbm.at[src_v.at[0]], rows_a, sem_g0)
        pltpu.async_copy(x_hbm.at[src_v.at[1]], rows_b, sem_g1)

        @pl.when((s < NS - 1) & (c == 0))
        def _():
            pltpu.sync_copy(x_hbm.at[pl.ds(r0, RPS)], acc_sh.at[pl.ds(r0, RPS)])

        @pl.when((s == NS - 1) & (c == 0))
        def _():
            pltpu.sync_copy(x_hbm.at[pl.ds(15 * RPS, RPS_LAST)],
                            acc_sh.at[pl.ds(15 * RPS, RPS_LAST)])

        @pl.when((s < NS - 1) & (c == 1))
        def _():
            pltpu.sync_copy(z_hbm.at[pl.ds(r0, RPS)], acc_sh.at[pl.ds(r0, RPS)])

        @pl.when((s == NS - 1) & (c == 1))
        def _():
            pltpu.sync_copy(z_hbm.at[pl.ds(15 * RPS, RPS_LAST)],
                            acc_sh.at[pl.ds(15 * RPS, RPS_LAST)])

        plsc.subcore_barrier()

        def body(k, carry):
            i = 2 * k
            pltpu.make_async_copy(d_hbm.at[wid, i], dst_a, sem_i0).wait()
            pltpu.make_async_copy(x_hbm.at[src_v.at[i]], rows_a, sem_g0).wait()
            pltpu.sync_copy(rows_a, acc_sh.at[dst_a.at[0]], add=True)

            @pl.when(i + 2 < NCHUNK)
            def _():
                pltpu.async_copy(d_hbm.at[wid, i + 2], dst_a, sem_i0)
                pltpu.async_copy(x_hbm.at[src_v.at[i + 2]], rows_a, sem_g0)

            pltpu.make_async_copy(d_hbm.at[wid, i + 1], dst_b, sem_i1).wait()
            pltpu.make_async_copy(x_hbm.at[src_v.at[i + 1]], rows_b,
                                  sem_g1).wait()
            pltpu.sync_copy(rows_b, acc_sh.at[dst_b.at[0]], add=True)

            @pl.when(i + 3 < NCHUNK)
            def _():
                pltpu.async_copy(d_hbm.at[wid, i + 3], dst_b, sem_i1)
                pltpu.async_copy(x_hbm.at[src_v.at[i + 3]], rows_b, sem_g1)

            return carry

        lax.fori_loop(0, NCHUNK // 2, body, 0, unroll=False)
        plsc.subcore_barrier()

        @pl.when(s < NS - 1)
        def _():
            pltpu.sync_copy(acc_sh.at[pl.ds(r0, RPS)],
                            out_hbm.at[c, pl.ds(r0, RPS)])

        @pl.when(s == NS - 1)
        def _():
            pltpu.sync_copy(acc_sh.at[pl.ds(15 * RPS, RPS_LAST)],
                            out_hbm.at[c, pl.ds(15 * RPS, RPS_LAST)])

    return agg(x, src_r, dst_r, zeros_nd)


_BN = 1000


def _tc_layer_body(a_ref, w1_ref, b1_ref, w2_ref, b2_ref, o_ref):
    h = a_ref[0] + a_ref[1]
    h = jnp.maximum(jnp.dot(h, w1_ref[...], preferred_element_type=jnp.float32)
                    + b1_ref[...], 0.0)
    h = jnp.maximum(jnp.dot(h, w2_ref[...], preferred_element_type=jnp.float32)
                    + b2_ref[...], 0.0)
    o_ref[...] = h


def _tc_final_body(a_ref, w1_ref, b1_ref, w2_ref, b2_ref,
                   l1w_ref, l1b_ref, l2w_ref, l2b_ref, o_ref):
    h = a_ref[0] + a_ref[1]
    h = jnp.maximum(jnp.dot(h, w1_ref[...], preferred_element_type=jnp.float32)
                    + b1_ref[...], 0.0)
    h = jnp.maximum(jnp.dot(h, w2_ref[...], preferred_element_type=jnp.float32)
                    + b2_ref[...], 0.0)
    h = jnp.maximum(jnp.dot(h, l1w_ref[...], preferred_element_type=jnp.float32)
                    + l1b_ref[...], 0.0)
    o_ref[...] = (jnp.dot(h, l2w_ref[...], preferred_element_type=jnp.float32)
                  + l2b_ref[...])


def _row_spec():
    return pl.BlockSpec((_BN, D), lambda i: (i, 0))


def _agg_spec():
    return pl.BlockSpec((NC, _BN, D), lambda i: (0, i, 0))


def _w_spec():
    return pl.BlockSpec((D, D), lambda i: (0, 0))


def _b_spec():
    return pl.BlockSpec((1, D), lambda i: (0, 0))


def _tc_layer(agg, w1, b1, w2, b2):
    return pl.pallas_call(
        _tc_layer_body,
        grid=(N // _BN,),
        in_specs=[_agg_spec(), _w_spec(), _b_spec(), _w_spec(), _b_spec()],
        out_specs=_row_spec(),
        out_shape=jax.ShapeDtypeStruct((N, D), jnp.float32),
    )(agg, w1, b1.reshape(1, D), w2, b2.reshape(1, D))


def _tc_final(agg, w1, b1, w2, b2, l1w, l1b, l2w, l2b):
    return pl.pallas_call(
        _tc_final_body,
        grid=(N // _BN,),
        in_specs=[_agg_spec(), _w_spec(), _b_spec(), _w_spec(), _b_spec(),
                  _w_spec(), _b_spec(), _w_spec(), _b_spec()],
        out_specs=_row_spec(),
        out_shape=jax.ShapeDtypeStruct((N, D), jnp.float32),
    )(agg, w1, b1.reshape(1, D), w2, b2.reshape(1, D),
      l1w, l1b.reshape(1, D), l2w, l2b.reshape(1, D))


def kernel(x, edge_index, c1w1, c1b1, c1w2, c1b2, c2w1, c2b1, c2w2, c2b2,
           c3w1, c3b1, c3w2, c3b2, l1w, l1b, l2w, l2b):
    ppw = EPW_PAD - EPW
    dummy_src = (jnp.arange(ppw, dtype=jnp.int32) * 41) % N
    src_r = jnp.concatenate(
        [edge_index[0].reshape(NW, EPW),
         jnp.broadcast_to(dummy_src[None, :], (NW, ppw))],
        axis=1).reshape(NW, NCHUNK, CH)
    sink = (N + jnp.arange(ppw, dtype=jnp.int32) % N_SINK)[None, :]
    dst_r = jnp.concatenate(
        [edge_index[1].reshape(NW, EPW),
         jnp.broadcast_to(sink, (NW, ppw))], axis=1).reshape(NW, NCHUNK, 1, CH)
    z = jnp.zeros((N, D), jnp.float32)
    a = _sc_aggregate(x, src_r, dst_r, z)
    h = _tc_layer(a, c1w1, c1b1, c1w2, c1b2)
    a = _sc_aggregate(h, src_r, dst_r, z)
    h = _tc_layer(a, c2w1, c2b1, c2w2, c2b2)
    a = _sc_aggregate(h, src_r, dst_r, z)
    return _tc_final(a, c3w1, c3b1, c3w2, c3b2, l1w, l1b, l2w, l2b)


# speedup vs baseline: 3.9396x; 1.0264x over previous
import functools

import jax
import jax.numpy as jnp
from jax import lax
from jax.experimental import pallas as pl
from jax.experimental.pallas import tpu as pltpu
from jax.experimental.pallas import tpu_sc as plsc

N = 10000
E = 320000
D = 128

NC = 2
NS = 16
NW = NC * NS
EPW = E // NW
CH = 128
NCHUNK = 80
EPW_PAD = NCHUNK * CH
E_PAD = NW * EPW_PAD
N_SINK = 256
N_ACC = N + N_SINK
RPS = 624
RPS_LAST = N - 15 * RPS


def _sc_aggregate(x, src_r, dst_r, zeros_nd):
    @functools.partial(
        pl.kernel,
        out_type=jax.ShapeDtypeStruct((NC, N, D), jnp.float32),
        mesh=plsc.VectorSubcoreMesh(core_axis_name="c", subcore_axis_name="s"),
        scratch_types=[
            pltpu.VMEM((NCHUNK, CH), jnp.int32),
            pltpu.VMEM((1, CH), jnp.int32),
            pltpu.VMEM((1, CH), jnp.int32),
            pltpu.VMEM((CH, D), jnp.float32),
            pltpu.VMEM((CH, D), jnp.float32),
            pltpu.VMEM_SHARED((N_ACC, D), jnp.float32),
            pltpu.SemaphoreType.DMA,
            pltpu.SemaphoreType.DMA,
            pltpu.SemaphoreType.DMA,
            pltpu.SemaphoreType.DMA,
        ],
    )
    def agg(x_hbm, s_hbm, d_hbm, z_hbm, out_hbm, src_v, dst_a, dst_b,
            rows_a, rows_b, acc_sh, sem_g0, sem_g1, sem_i0, sem_i1):
        c = lax.axis_index("c")
        s = lax.axis_index("s")
        wid = s * NC + c
        r0 = s * RPS

        pltpu.sync_copy(s_hbm.at[wid], src_v)
        pltpu.async_copy(d_hbm.at[wid, 0], dst_a, sem_i0)
        pltpu.async_copy(d_hbm.at[wid, 1], dst_b, sem_i1)
        pltpu.async_copy(x_hbm.at[src_v.at[0]], rows_a, sem_g0)
        pltpu.async_copy(x_hbm.at[src_v.at[1]], rows_b, sem_g1)

        @pl.when((s < NS - 1) & (c == 0))
        def _():
            pltpu.sync_copy(x_hbm.at[pl.ds(r0, RPS)], acc_sh.at[pl.ds(r0, RPS)])

        @pl.when((s == NS - 1) & (c == 0))
        def _():
            pltpu.sync_copy(x_hbm.at[pl.ds(15 * RPS, RPS_LAST)],
                            acc_sh.at[pl.ds(15 * RPS, RPS_LAST)])

        @pl.when((s < NS - 1) & (c == 1))
        def _():
            pltpu.sync_copy(z_hbm.at[pl.ds(r0, RPS)], acc_sh.at[pl.ds(r0, RPS)])

        @pl.when((s == NS - 1) & (c == 1))
        def _():
            pltpu.sync_copy(z_hbm.at[pl.ds(15 * RPS, RPS_LAST)],
                            acc_sh.at[pl.ds(15 * RPS, RPS_LAST)])

        plsc.subcore_barrier()

        def body(k, carry):
            i = 2 * k
            pltpu.make_async_copy(d_hbm.at[wid, i], dst_a, sem_i0).wait()
            pltpu.make_async_copy(x_hbm.at[src_v.at[i]], rows_a, sem_g0).wait()
            pltpu.sync_copy(rows_a, acc_sh.at[dst_a.at[0]], add=True)

            @pl.when(i + 2 < NCHUNK)
            def _():
                pltpu.async_copy(d_hbm.at[wid, i + 2], dst_a, sem_i0)
                pltpu.async_copy(x_hbm.at[src_v.at[i + 2]], rows_a, sem_g0)

            pltpu.make_async_copy(d_hbm.at[wid, i + 1], dst_b, sem_i1).wait()
            pltpu.make_async_copy(x_hbm.at[src_v.at[i + 1]], rows_b,
                                  sem_g1).wait()
            pltpu.sync_copy(rows_b, acc_sh.at[dst_b.at[0]], add=True)

            @pl.when(i + 3 < NCHUNK)
            def _():
                pltpu.async_copy(d_hbm.at[wid, i + 3], dst_b, sem_i1)
                pltpu.async_copy(x_hbm.at[src_v.at[i + 3]], rows_b, sem_g1)

            return carry

        lax.fori_loop(0, NCHUNK // 2, body, 0, unroll=False)
        plsc.subcore_barrier()

        @pl.when(s < NS - 1)
        def _():
            pltpu.sync_copy(acc_sh.at[pl.ds(r0, RPS)],
                            out_hbm.at[c, pl.ds(r0, RPS)])

        @pl.when(s == NS - 1)
        def _():
            pltpu.sync_copy(acc_sh.at[pl.ds(15 * RPS, RPS_LAST)],
                            out_hbm.at[c, pl.ds(15 * RPS, RPS_LAST)])

    return agg(x, src_r, dst_r, zeros_nd)


_BN = 2000


def _tc_layer_body(a_ref, w1_ref, b1_ref, w2_ref, b2_ref, o_ref):
    h = a_ref[0] + a_ref[1]
    h = jnp.maximum(jnp.dot(h, w1_ref[...], preferred_element_type=jnp.float32)
                    + b1_ref[...], 0.0)
    h = jnp.maximum(jnp.dot(h, w2_ref[...], preferred_element_type=jnp.float32)
                    + b2_ref[...], 0.0)
    o_ref[...] = h


def _tc_final_body(a_ref, w1_ref, b1_ref, w2_ref, b2_ref,
                   l1w_ref, l1b_ref, l2w_ref, l2b_ref, o_ref):
    h = a_ref[0] + a_ref[1]
    h = jnp.maximum(jnp.dot(h, w1_ref[...], preferred_element_type=jnp.float32)
                    + b1_ref[...], 0.0)
    h = jnp.maximum(jnp.dot(h, w2_ref[...], preferred_element_type=jnp.float32)
                    + b2_ref[...], 0.0)
    h = jnp.maximum(jnp.dot(h, l1w_ref[...], preferred_element_type=jnp.float32)
                    + l1b_ref[...], 0.0)
    o_ref[...] = (jnp.dot(h, l2w_ref[...], preferred_element_type=jnp.float32)
                  + l2b_ref[...])


def _row_spec():
    return pl.BlockSpec((_BN, D), lambda i: (i, 0))


def _agg_spec():
    return pl.BlockSpec((NC, _BN, D), lambda i: (0, i, 0))


def _w_spec():
    return pl.BlockSpec((D, D), lambda i: (0, 0))


def _b_spec():
    return pl.BlockSpec((1, D), lambda i: (0, 0))


def _tc_layer(agg, w1, b1, w2, b2):
    return pl.pallas_call(
        _tc_layer_body,
        grid=(N // _BN,),
        in_specs=[_agg_spec(), _w_spec(), _b_spec(), _w_spec(), _b_spec()],
        out_specs=_row_spec(),
        out_shape=jax.ShapeDtypeStruct((N, D), jnp.float32),
    )(agg, w1, b1.reshape(1, D), w2, b2.reshape(1, D))


def _tc_final(agg, w1, b1, w2, b2, l1w, l1b, l2w, l2b):
    return pl.pallas_call(
        _tc_final_body,
        grid=(N // _BN,),
        in_specs=[_agg_spec(), _w_spec(), _b_spec(), _w_spec(), _b_spec(),
                  _w_spec(), _b_spec(), _w_spec(), _b_spec()],
        out_specs=_row_spec(),
        out_shape=jax.ShapeDtypeStruct((N, D), jnp.float32),
    )(agg, w1, b1.reshape(1, D), w2, b2.reshape(1, D),
      l1w, l1b.reshape(1, D), l2w, l2b.reshape(1, D))


def kernel(x, edge_index, c1w1, c1b1, c1w2, c1b2, c2w1, c2b1, c2w2, c2b2,
           c3w1, c3b1, c3w2, c3b2, l1w, l1b, l2w, l2b):
    ppw = EPW_PAD - EPW
    dummy_src = (jnp.arange(ppw, dtype=jnp.int32) * 41) % N
    src_r = jnp.concatenate(
        [edge_index[0].reshape(NW, EPW),
         jnp.broadcast_to(dummy_src[None, :], (NW, ppw))],
        axis=1).reshape(NW, NCHUNK, CH)
    sink = (N + jnp.arange(ppw, dtype=jnp.int32) % N_SINK)[None, :]
    dst_r = jnp.concatenate(
        [edge_index[1].reshape(NW, EPW),
         jnp.broadcast_to(sink, (NW, ppw))], axis=1).reshape(NW, NCHUNK, 1, CH)
    z = jnp.zeros((N, D), jnp.float32)
    a = _sc_aggregate(x, src_r, dst_r, z)
    h = _tc_layer(a, c1w1, c1b1, c1w2, c1b2)
    a = _sc_aggregate(h, src_r, dst_r, z)
    h = _tc_layer(a, c2w1, c2b1, c2w2, c2b2)
    a = _sc_aggregate(h, src_r, dst_r, z)
    return _tc_final(a, c3w1, c3b1, c3w2, c3b2, l1w, l1b, l2w, l2b)


# speedup vs baseline: 3.9552x; 1.0040x over previous
import functools

import jax
import jax.numpy as jnp
from jax import lax
from jax.experimental import pallas as pl
from jax.experimental.pallas import tpu as pltpu
from jax.experimental.pallas import tpu_sc as plsc

N = 10000
E = 320000
D = 128

NC = 2
NS = 16
NW = NC * NS
EPW = E // NW
CH = 128
NCHUNK = 80
EPW_PAD = NCHUNK * CH
E_PAD = NW * EPW_PAD
N_SINK = 256
N_ACC = N + N_SINK
RPS = 624
RPS_LAST = N - 15 * RPS


def _sc_aggregate(x, src_r, dst_r, zeros_nd):
    @functools.partial(
        pl.kernel,
        out_type=jax.ShapeDtypeStruct((NC, N, D), jnp.float32),
        mesh=plsc.VectorSubcoreMesh(core_axis_name="c", subcore_axis_name="s"),
        scratch_types=[
            pltpu.VMEM((NCHUNK, CH), jnp.int32),
            pltpu.VMEM((1, CH), jnp.int32),
            pltpu.VMEM((1, CH), jnp.int32),
            pltpu.VMEM((CH, D), jnp.float32),
            pltpu.VMEM((CH, D), jnp.float32),
            pltpu.VMEM_SHARED((N_ACC, D), jnp.float32),
            pltpu.SemaphoreType.DMA,
            pltpu.SemaphoreType.DMA,
            pltpu.SemaphoreType.DMA,
            pltpu.SemaphoreType.DMA,
        ],
    )
    def agg(x_hbm, s_hbm, d_hbm, z_hbm, out_hbm, src_v, dst_a, dst_b,
            rows_a, rows_b, acc_sh, sem_g0, sem_g1, sem_i0, sem_i1):
        c = lax.axis_index("c")
        s = lax.axis_index("s")
        wid = s * NC + c
        r0 = s * RPS

        pltpu.sync_copy(s_hbm.at[wid], src_v)
        pltpu.async_copy(d_hbm.at[wid, 0], dst_a, sem_i0)
        pltpu.async_copy(d_hbm.at[wid, 1], dst_b, sem_i1)
        pltpu.async_copy(x_hbm.at[src_v.at[0]], rows_a, sem_g0)
        pltpu.async_copy(x_hbm.at[src_v.at[1]], rows_b, sem_g1)

        @pl.when((s < NS - 1) & (c == 0))
        def _():
            pltpu.sync_copy(x_hbm.at[pl.ds(r0, RPS)], acc_sh.at[pl.ds(r0, RPS)])

        @pl.when((s == NS - 1) & (c == 0))
        def _():
            pltpu.sync_copy(x_hbm.at[pl.ds(15 * RPS, RPS_LAST)],
                            acc_sh.at[pl.ds(15 * RPS, RPS_LAST)])

        @pl.when((s < NS - 1) & (c == 1))
        def _():
            pltpu.sync_copy(z_hbm.at[pl.ds(r0, RPS)], acc_sh.at[pl.ds(r0, RPS)])

        @pl.when((s == NS - 1) & (c == 1))
        def _():
            pltpu.sync_copy(z_hbm.at[pl.ds(15 * RPS, RPS_LAST)],
                            acc_sh.at[pl.ds(15 * RPS, RPS_LAST)])

        plsc.subcore_barrier()

        def body(k, carry):
            i = 2 * k
            pltpu.make_async_copy(d_hbm.at[wid, i], dst_a, sem_i0).wait()
            pltpu.make_async_copy(x_hbm.at[src_v.at[i]], rows_a, sem_g0).wait()
            pltpu.sync_copy(rows_a, acc_sh.at[dst_a.at[0]], add=True)

            @pl.when(i + 2 < NCHUNK)
            def _():
                pltpu.async_copy(d_hbm.at[wid, i + 2], dst_a, sem_i0)
                pltpu.async_copy(x_hbm.at[src_v.at[i + 2]], rows_a, sem_g0)

            pltpu.make_async_copy(d_hbm.at[wid, i + 1], dst_b, sem_i1).wait()
            pltpu.make_async_copy(x_hbm.at[src_v.at[i + 1]], rows_b,
                                  sem_g1).wait()
            pltpu.sync_copy(rows_b, acc_sh.at[dst_b.at[0]], add=True)

            @pl.when(i + 3 < NCHUNK)
            def _():
                pltpu.async_copy(d_hbm.at[wid, i + 3], dst_b, sem_i1)
                pltpu.async_copy(x_hbm.at[src_v.at[i + 3]], rows_b, sem_g1)

            return carry

        lax.fori_loop(0, NCHUNK // 2, body, 0, unroll=False)
        plsc.subcore_barrier()

        @pl.when(s < NS - 1)
        def _():
            pltpu.sync_copy(acc_sh.at[pl.ds(r0, RPS)],
                            out_hbm.at[c, pl.ds(r0, RPS)])

        @pl.when(s == NS - 1)
        def _():
            pltpu.sync_copy(acc_sh.at[pl.ds(15 * RPS, RPS_LAST)],
                            out_hbm.at[c, pl.ds(15 * RPS, RPS_LAST)])

    return agg(x, src_r, dst_r, zeros_nd)


_BN = 5000


def _tc_layer_body(a_ref, w1_ref, b1_ref, w2_ref, b2_ref, o_ref):
    h = a_ref[0] + a_ref[1]
    h = jnp.maximum(jnp.dot(h, w1_ref[...], preferred_element_type=jnp.float32)
                    + b1_ref[...], 0.0)
    h = jnp.maximum(jnp.dot(h, w2_ref[...], preferred_element_type=jnp.float32)
                    + b2_ref[...], 0.0)
    o_ref[...] = h


def _tc_final_body(a_ref, w1_ref, b1_ref, w2_ref, b2_ref,
                   l1w_ref, l1b_ref, l2w_ref, l2b_ref, o_ref):
    h = a_ref[0] + a_ref[1]
    h = jnp.maximum(jnp.dot(h, w1_ref[...], preferred_element_type=jnp.float32)
                    + b1_ref[...], 0.0)
    h = jnp.maximum(jnp.dot(h, w2_ref[...], preferred_element_type=jnp.float32)
                    + b2_ref[...], 0.0)
    h = jnp.maximum(jnp.dot(h, l1w_ref[...], preferred_element_type=jnp.float32)
                    + l1b_ref[...], 0.0)
    o_ref[...] = (jnp.dot(h, l2w_ref[...], preferred_element_type=jnp.float32)
                  + l2b_ref[...])


def _row_spec():
    return pl.BlockSpec((_BN, D), lambda i: (i, 0))


def _agg_spec():
    return pl.BlockSpec((NC, _BN, D), lambda i: (0, i, 0))


def _w_spec():
    return pl.BlockSpec((D, D), lambda i: (0, 0))


def _b_spec():
    return pl.BlockSpec((1, D), lambda i: (0, 0))


def _tc_layer(agg, w1, b1, w2, b2):
    return pl.pallas_call(
        _tc_layer_body,
        grid=(N // _BN,),
        in_specs=[_agg_spec(), _w_spec(), _b_spec(), _w_spec(), _b_spec()],
        out_specs=_row_spec(),
        out_shape=jax.ShapeDtypeStruct((N, D), jnp.float32),
    )(agg, w1, b1.reshape(1, D), w2, b2.reshape(1, D))


def _tc_final(agg, w1, b1, w2, b2, l1w, l1b, l2w, l2b):
    return pl.pallas_call(
        _tc_final_body,
        grid=(N // _BN,),
        in_specs=[_agg_spec(), _w_spec(), _b_spec(), _w_spec(), _b_spec(),
                  _w_spec(), _b_spec(), _w_spec(), _b_spec()],
        out_specs=_row_spec(),
        out_shape=jax.ShapeDtypeStruct((N, D), jnp.float32),
    )(agg, w1, b1.reshape(1, D), w2, b2.reshape(1, D),
      l1w, l1b.reshape(1, D), l2w, l2b.reshape(1, D))


def kernel(x, edge_index, c1w1, c1b1, c1w2, c1b2, c2w1, c2b1, c2w2, c2b2,
           c3w1, c3b1, c3w2, c3b2, l1w, l1b, l2w, l2b):
    ppw = EPW_PAD - EPW
    dummy_src = (jnp.arange(ppw, dtype=jnp.int32) * 41) % N
    src_r = jnp.concatenate(
        [edge_index[0].reshape(NW, EPW),
         jnp.broadcast_to(dummy_src[None, :], (NW, ppw))],
        axis=1).reshape(NW, NCHUNK, CH)
    sink = (N + jnp.arange(ppw, dtype=jnp.int32) % N_SINK)[None, :]
    dst_r = jnp.concatenate(
        [edge_index[1].reshape(NW, EPW),
         jnp.broadcast_to(sink, (NW, ppw))], axis=1).reshape(NW, NCHUNK, 1, CH)
    z = jnp.zeros((N, D), jnp.float32)
    a = _sc_aggregate(x, src_r, dst_r, z)
    h = _tc_layer(a, c1w1, c1b1, c1w2, c1b2)
    a = _sc_aggregate(h, src_r, dst_r, z)
    h = _tc_layer(a, c2w1, c2b1, c2w2, c2b2)
    a = _sc_aggregate(h, src_r, dst_r, z)
    return _tc_final(a, c3w1, c3b1, c3w2, c3b2, l1w, l1b, l2w, l2b)
